# Initial kernel scaffold; baseline (speedup 1.0000x reference)
import jax
import jax.numpy as jnp
from jax.experimental import pallas as pl


def kernel(x, vq_head_weights, vq_codebooks, temperature):
    raise NotImplementedError("write your pallas kernel here")


# speedup vs baseline: 2.8820x; 2.8820x over previous
import jax
import jax.numpy as jnp
from jax.experimental import pallas as pl
from jax.experimental.pallas import tpu as pltpu

_N_EMBD = 1024
_N_HEADS = 32
_N_OPTS = 1024
_HEAD = _N_EMBD // _N_HEADS

_T = 256


def _vq_block_kernel(x_ref, w_ref, c_ref, o_ref):
    for h in range(_N_HEADS):
        xh = x_ref[:, h * _HEAD:(h + 1) * _HEAD]
        wh = w_ref[h]
        logits = jax.lax.dot_general(
            xh, wh, (((1,), (0,)), ((), ())),
            preferred_element_type=jnp.float32)
        m = jnp.max(logits, axis=1, keepdims=True)
        e = jnp.exp(logits - m)
        s = jnp.sum(e, axis=1, keepdims=True)
        acc = jax.lax.dot_general(
            e, c_ref[h], (((1,), (1,)), ((), ())),
            preferred_element_type=jnp.float32)
        o_ref[:, h * _HEAD:(h + 1) * _HEAD] = acc / s


def kernel(x, vq_head_weights, vq_codebooks, temperature):
    B, S, _ = x.shape
    rows = B * S
    x2 = x.reshape(rows, _N_EMBD)
    w = jnp.swapaxes(vq_head_weights / temperature, 1, 2)
    c = jnp.swapaxes(vq_codebooks, 1, 2)

    grid = (rows // _T,)
    out = pl.pallas_call(
        _vq_block_kernel,
        grid=grid,
        in_specs=[
            pl.BlockSpec((_T, _N_EMBD), lambda r: (r, 0)),
            pl.BlockSpec((_N_HEADS, _HEAD, _N_OPTS), lambda r: (0, 0, 0)),
            pl.BlockSpec((_N_HEADS, _HEAD, _N_OPTS), lambda r: (0, 0, 0)),
        ],
        out_specs=pl.BlockSpec((_T, _N_EMBD), lambda r: (r, 0)),
        out_shape=jax.ShapeDtypeStruct((rows, _N_EMBD), jnp.float32),
    )(x2, w, c)
    return out.reshape(B, S, _N_EMBD)
